# Initial kernel scaffold; baseline (speedup 1.0000x reference)
import functools

import jax
import jax.numpy as jnp
from jax import lax
from jax.experimental import pallas as pl
from jax.experimental.pallas import tpu as pltpu
from jax.experimental.pallas import tpu_sc as plsc

N = 10000
E = 320000
D = 128
NLAYER = 3

NC = 2
NS = 16
NW = NC * NS

K = 128
CH = -(-E // (NW * K))
EPT = CH * K
E_PAD = EPT * NW

NP = 10240
RPT = NP // NS
BLK = 256


_sc_mesh = plsc.VectorSubcoreMesh(
    core_axis_name="c", subcore_axis_name="s", num_cores=NC, num_subcores=NS)


@functools.partial(
    pl.kernel,
    out_type=jax.ShapeDtypeStruct((NC, NP, D), jnp.float32),
    mesh=_sc_mesh,
    scratch_types=[
        pltpu.VMEM((2, K), jnp.int32),
        pltpu.VMEM((K, D), jnp.float32),
        pltpu.VMEM_SHARED((NP, D), jnp.float32),
        pltpu.SemaphoreType.DMA,
    ],
)
def _sc_agg(h_hbm, ei_hbm, zeros_hbm, out_hbm, ebuf, rows, acc, sem):
    cid = lax.axis_index("c")
    sid = lax.axis_index("s")
    tid = sid * NC + cid

    pltpu.sync_copy(zeros_hbm.at[pl.ds(sid * RPT, RPT)],
                    acc.at[pl.ds(sid * RPT, RPT)])
    plsc.subcore_barrier()

    def body(i, carry):
        pltpu.sync_copy(ei_hbm.at[tid, i], ebuf)
        pltpu.async_copy(h_hbm.at[ebuf.at[0]], rows, sem).wait()
        pltpu.sync_copy(rows, acc.at[ebuf.at[1]], add=True)
        return carry

    lax.fori_loop(0, CH, body, 0)
    plsc.subcore_barrier()

    pltpu.sync_copy(acc.at[pl.ds(sid * RPT, RPT)],
                    out_hbm.at[cid, pl.ds(sid * RPT, RPT)])


def _ln(x, g, b):
    mu = jnp.mean(x, axis=-1, keepdims=True)
    var = jnp.mean((x - mu) * (x - mu), axis=-1, keepdims=True)
    return (x - mu) * lax.rsqrt(var + 1e-5) * g + b


def _mlp_body_inner(h_ref, a0_ref, a1_ref, w1_ref, b1_ref, w2_ref, b2_ref,
                    ng_ref, nb_ref, sg_ref, sb_ref, eps_ref, out_ref):
    h = h_ref[...]
    z = (1.0 + eps_ref[0]) * h + a0_ref[...] + a1_ref[...]
    z = jnp.maximum(
        jnp.dot(z, w1_ref[...], preferred_element_type=jnp.float32)
        + b1_ref[...], 0.0)
    z = jnp.dot(z, w2_ref[...], preferred_element_type=jnp.float32) + b2_ref[...]
    z = _ln(z, ng_ref[...], nb_ref[...])
    z = jnp.maximum(_ln(z, sg_ref[...], sb_ref[...]), 0.0)
    out_ref[...] = z + h


def _mlp_body_last(h_ref, a0_ref, a1_ref, w1_ref, b1_ref, w2_ref, b2_ref,
                   ng_ref, nb_ref, eps_ref, out_ref):
    h = h_ref[...]
    z = (1.0 + eps_ref[0]) * h + a0_ref[...] + a1_ref[...]
    z = jnp.maximum(
        jnp.dot(z, w1_ref[...], preferred_element_type=jnp.float32)
        + b1_ref[...], 0.0)
    z = jnp.dot(z, w2_ref[...], preferred_element_type=jnp.float32) + b2_ref[...]
    z = _ln(z, ng_ref[...], nb_ref[...])
    out_ref[...] = z + h


def _row_spec():
    return pl.BlockSpec((BLK, D), lambda i: (i, 0))


def _full_spec(shape):
    nd = len(shape)
    return pl.BlockSpec(shape, lambda i: (0,) * nd)


def _tc_mlp(inner, h, a0, a1, w1, b1, w2, b2, ng, nb, sg, sb, eps):
    vecs = [v.reshape(1, D) for v in (b1, b2, ng, nb)]
    body = _mlp_body_last
    if inner:
        vecs += [sg.reshape(1, D), sb.reshape(1, D)]
        body = _mlp_body_inner
    in_specs = (
        [_row_spec(), _row_spec(), _row_spec(),
         _full_spec((D, D)), _full_spec((1, D)),
         _full_spec((D, D)), _full_spec((1, D)),
         _full_spec((1, D)), _full_spec((1, D))]
        + ([_full_spec((1, D)), _full_spec((1, D))] if inner else [])
        + [pl.BlockSpec(memory_space=pltpu.SMEM)]
    )
    return pl.pallas_call(
        body,
        grid=(NP // BLK,),
        in_specs=in_specs,
        out_specs=_row_spec(),
        out_shape=jax.ShapeDtypeStruct((NP, D), jnp.float32),
    )(h, a0, a1, w1, vecs[0], w2, vecs[1], vecs[2], vecs[3],
      *(vecs[4:] if inner else []), eps)


def kernel(x, edge_index, params):
    src = edge_index[0].astype(jnp.int32)
    dst = edge_index[1].astype(jnp.int32)
    pad = E_PAD - E
    if pad:
        src = jnp.concatenate([src, jnp.zeros((pad,), jnp.int32)])
        dst = jnp.concatenate([dst, jnp.full((pad,), N, jnp.int32)])
    ei = jnp.stack(
        [src.reshape(NW, CH, K), dst.reshape(NW, CH, K)], axis=2)
    zeros_rows = jnp.zeros((NP, D), jnp.float32)
    h = jnp.zeros((NP, D), jnp.float32).at[:N].set(x)
    for l in range(NLAYER):
        agg = _sc_agg(h, ei, zeros_rows)
        inner = l < NLAYER - 1
        h = _tc_mlp(
            inner, h, agg[0], agg[1],
            params[f'W1_{l}'], params[f'b1_{l}'],
            params[f'W2_{l}'], params[f'b2_{l}'],
            params[f'ng_{l}'], params[f'nb_{l}'],
            params[f'sg_{l}'] if inner else None,
            params[f'sb_{l}'] if inner else None,
            params[f'eps_{l}'])
    return h[:N]


# speedup vs baseline: 3.9976x; 3.9976x over previous
import functools

import jax
import jax.numpy as jnp
from jax import lax
from jax.experimental import pallas as pl
from jax.experimental.pallas import tpu as pltpu
from jax.experimental.pallas import tpu_sc as plsc

N = 10000
E = 320000
D = 128
NLAYER = 3

NC = 2
NS = 16
NW = NC * NS

K = 128
CH = -(-E // (NW * K))
EPT = CH * K
E_PAD = EPT * NW

NP = 10240
RPT = NP // NS
BLK = 256


def _sc_agg_body(h_hbm, ei_hbm, zeros_hbm, out_hbm, ebuf, rows, acc, sem):
    cid = lax.axis_index("c")
    sid = lax.axis_index("s")
    tid = sid * NC + cid

    pltpu.sync_copy(zeros_hbm.at[pl.ds(sid * RPT, RPT)],
                    acc.at[pl.ds(sid * RPT, RPT)])
    plsc.subcore_barrier()

    def body(i, carry):
        pltpu.sync_copy(ei_hbm.at[tid, i], ebuf)
        pltpu.async_copy(h_hbm.at[ebuf.at[0]], rows, sem).wait()
        pltpu.sync_copy(rows, acc.at[ebuf.at[1]], add=True)
        return carry

    lax.fori_loop(0, CH, body, 0)
    plsc.subcore_barrier()

    pltpu.sync_copy(acc.at[pl.ds(sid * RPT, RPT)],
                    out_hbm.at[cid, pl.ds(sid * RPT, RPT)])


@functools.cache
def _sc_agg():
    mesh = plsc.VectorSubcoreMesh(
        core_axis_name="c", subcore_axis_name="s",
        num_cores=NC, num_subcores=NS)
    return pl.kernel(
        _sc_agg_body,
        out_type=jax.ShapeDtypeStruct((NC, NP, D), jnp.float32),
        mesh=mesh,
        scratch_types=[
            pltpu.VMEM((2, K), jnp.int32),
            pltpu.VMEM((K, D), jnp.float32),
            pltpu.VMEM_SHARED((NP, D), jnp.float32),
            pltpu.SemaphoreType.DMA,
        ],
    )


def _ln(x, g, b):
    mu = jnp.mean(x, axis=-1, keepdims=True)
    var = jnp.mean((x - mu) * (x - mu), axis=-1, keepdims=True)
    return (x - mu) * lax.rsqrt(var + 1e-5) * g + b


def _mlp_body_inner(h_ref, a0_ref, a1_ref, w1_ref, b1_ref, w2_ref, b2_ref,
                    ng_ref, nb_ref, sg_ref, sb_ref, eps_ref, out_ref):
    h = h_ref[...]
    z = (1.0 + eps_ref[0]) * h + a0_ref[...] + a1_ref[...]
    z = jnp.maximum(
        jnp.dot(z, w1_ref[...], preferred_element_type=jnp.float32)
        + b1_ref[...], 0.0)
    z = jnp.dot(z, w2_ref[...], preferred_element_type=jnp.float32) + b2_ref[...]
    z = _ln(z, ng_ref[...], nb_ref[...])
    z = jnp.maximum(_ln(z, sg_ref[...], sb_ref[...]), 0.0)
    out_ref[...] = z + h


def _mlp_body_last(h_ref, a0_ref, a1_ref, w1_ref, b1_ref, w2_ref, b2_ref,
                   ng_ref, nb_ref, eps_ref, out_ref):
    h = h_ref[...]
    z = (1.0 + eps_ref[0]) * h + a0_ref[...] + a1_ref[...]
    z = jnp.maximum(
        jnp.dot(z, w1_ref[...], preferred_element_type=jnp.float32)
        + b1_ref[...], 0.0)
    z = jnp.dot(z, w2_ref[...], preferred_element_type=jnp.float32) + b2_ref[...]
    z = _ln(z, ng_ref[...], nb_ref[...])
    out_ref[...] = z + h


def _row_spec():
    return pl.BlockSpec((BLK, D), lambda i: (i, 0))


def _full_spec(shape):
    nd = len(shape)
    return pl.BlockSpec(shape, lambda i: (0,) * nd)


def _tc_mlp(inner, h, a0, a1, w1, b1, w2, b2, ng, nb, sg, sb, eps):
    vecs = [v.reshape(1, D) for v in (b1, b2, ng, nb)]
    body = _mlp_body_last
    if inner:
        vecs += [sg.reshape(1, D), sb.reshape(1, D)]
        body = _mlp_body_inner
    in_specs = (
        [_row_spec(), _row_spec(), _row_spec(),
         _full_spec((D, D)), _full_spec((1, D)),
         _full_spec((D, D)), _full_spec((1, D)),
         _full_spec((1, D)), _full_spec((1, D))]
        + ([_full_spec((1, D)), _full_spec((1, D))] if inner else [])
        + [pl.BlockSpec(memory_space=pltpu.SMEM)]
    )
    return pl.pallas_call(
        body,
        grid=(NP // BLK,),
        in_specs=in_specs,
        out_specs=_row_spec(),
        out_shape=jax.ShapeDtypeStruct((NP, D), jnp.float32),
    )(h, a0, a1, w1, vecs[0], w2, vecs[1], vecs[2], vecs[3],
      *(vecs[4:] if inner else []), eps)


def kernel(x, edge_index, params):
    src = edge_index[0].astype(jnp.int32)
    dst = edge_index[1].astype(jnp.int32)
    pad = E_PAD - E
    if pad:
        src = jnp.concatenate([src, jnp.zeros((pad,), jnp.int32)])
        dst = jnp.concatenate([dst, jnp.full((pad,), N, jnp.int32)])
    ei = jnp.stack(
        [src.reshape(NW, CH, K), dst.reshape(NW, CH, K)], axis=2)
    zeros_rows = jnp.zeros((NP, D), jnp.float32)
    h = jnp.zeros((NP, D), jnp.float32).at[:N].set(x)
    for l in range(NLAYER):
        agg = _sc_agg()(h, ei, zeros_rows)
        inner = l < NLAYER - 1
        h = _tc_mlp(
            inner, h, agg[0], agg[1],
            params[f'W1_{l}'], params[f'b1_{l}'],
            params[f'W2_{l}'], params[f'b2_{l}'],
            params[f'ng_{l}'], params[f'nb_{l}'],
            params[f'sg_{l}'] if inner else None,
            params[f'sb_{l}'] if inner else None,
            params[f'eps_{l}'])
    return h[:N]
